# Initial kernel scaffold; baseline (speedup 1.0000x reference)
import jax
import jax.numpy as jnp
from jax.experimental import pallas as pl


def kernel(x, g_ij, idx_i, idx_j, n_atoms, rand_vec, W_g, b_g, W_j, b_j, W_i, b_i, int_res_W1, int_res_b1, int_res_W2, int_res_b2, W_f, b_f, atom_res_W1, atom_res_b1, atom_res_W2, atom_res_b2, out_res_W1, out_res_b1, out_res_W2, out_res_b2, W_out, b_out):
    raise NotImplementedError("write your pallas kernel here")


# speedup vs baseline: 2.7024x; 2.7024x over previous
import functools

import jax
import jax.numpy as jnp
from jax import lax
from jax.experimental import pallas as pl
from jax.experimental.pallas import tpu as pltpu
from jax.experimental.pallas import tpu_sc as plsc

_N = 10000
_E = 320000
_D = 128

_NC = 2
_NS = 16
_NW = _NC * _NS
_EPW = _E // _NW
_C = 80
_NCHUNK = _EPW // _C
_RPS = 640
_NP = _NS * _RPS
_ZR = 128

_BN = 2000
_BE = 6400


def _silu(t):
    return t / (1.0 + jnp.exp(-t))


def _pre_body(x_ref, wj_ref, bj_ref, wi_ref, bi_ref, u_ref, vm_ref):
    xb = x_ref[...]
    sx = _silu(xb)
    u_ref[...] = _silu(sx @ wj_ref[...] + bj_ref[...])
    vm_ref[...] = _silu(sx @ wi_ref[...] + bi_ref[...])


def _gmat_body(g_ref, wg_ref, bg_ref, o_ref):
    o_ref[...] = g_ref[...] @ wg_ref[...] + bg_ref[...]


def _res_block(h, w1, b1, w2, b2):
    y = _silu(h) @ w1 + b1
    y = _silu(y) @ w2 + b2
    return y + h


def _post_body(n_int, n_atom, n_out, *refs):
    it = iter(refs)
    x_ref, v0_ref, v1_ref, vm_ref, rv_ref = (next(it) for _ in range(5))
    int_w = [tuple(next(it)[...] for _ in range(4)) for _ in range(n_int)]
    wf, bf = next(it)[...], next(it)[...]
    atom_w = [tuple(next(it)[...] for _ in range(4)) for _ in range(n_atom)]
    out_w = [tuple(next(it)[...] for _ in range(4)) for _ in range(n_out)]
    wout, bout = next(it)[...], next(it)[...]
    out_ref, x1_ref = next(it), next(it)

    xb = x_ref[...]
    v = v0_ref[...] + v1_ref[...] + vm_ref[...]
    for w1, b1, w2, b2 in int_w:
        v = _res_block(v, w1, b1, w2, b2)
    v = _silu(v)
    x1 = rv_ref[...] * xb + v @ wf + bf
    for w1, b1, w2, b2 in atom_w:
        x1 = _res_block(x1, w1, b1, w2, b2)
    x1_ref[...] = x1
    h = x1
    for w1, b1, w2, b2 in out_w:
        h = _res_block(h, w1, b1, w2, b2)
    out_ref[...] = _silu(h) @ wout + bout


def _make_sc_kernel(n_nodes):
    mesh = plsc.VectorSubcoreMesh(core_axis_name="c", subcore_axis_name="s")

    @functools.partial(
        pl.kernel,
        mesh=mesh,
        out_type=jax.ShapeDtypeStruct((_NC * _NP, _D), jnp.float32),
        scratch_types=[
            pltpu.VMEM((_C,), jnp.int32),
            pltpu.VMEM((_C,), jnp.int32),
            pltpu.VMEM((_C, _D), jnp.float32),
            pltpu.VMEM((_C, _D), jnp.float32),
            pltpu.VMEM((_ZR, _D), jnp.float32),
            pltpu.VMEM_SHARED((_NP, _D), jnp.float32),
            pltpu.SemaphoreType.DMA,
        ],
    )
    def sc_fn(u_hbm, g_hbm, idxi_hbm, idxj_hbm, out_hbm,
              idxi_v, idxj_v, g_v, u_v, z_v, v_sh, sem):
        cid = lax.axis_index("c")
        sid = lax.axis_index("s")
        w = cid * _NS + sid

        def zrow(r, carry):
            for l in range(_D // 16):
                z_v[r, pl.ds(l * 16, 16)] = jnp.zeros((16,), jnp.float32)
            return carry
        lax.fori_loop(0, _ZR, zrow, 0)
        for j in range(_RPS // _ZR):
            pltpu.sync_copy(z_v, v_sh.at[pl.ds(sid * _RPS + j * _ZR, _ZR)])
        plsc.subcore_barrier()

        def chunk(k, carry):
            base = w * _EPW + k * _C
            pltpu.sync_copy(idxi_hbm.at[pl.ds(base, _C)], idxi_v)
            pltpu.sync_copy(idxj_hbm.at[pl.ds(base, _C)], idxj_v)
            pltpu.async_copy(u_hbm.at[idxj_v], u_v, sem).wait()
            pltpu.sync_copy(g_hbm.at[pl.ds(base, _C)], g_v)

            def mrow(r, c2):
                for l in range(_D // 16):
                    sl = pl.ds(l * 16, 16)
                    u_v[r, sl] = u_v[r, sl] * g_v[r, sl]
                return c2
            lax.fori_loop(0, _C, mrow, 0)
            pltpu.sync_copy(u_v, v_sh.at[idxi_v], add=True)
            return carry
        lax.fori_loop(0, _NCHUNK, chunk, 0)

        plsc.subcore_barrier()
        pltpu.sync_copy(
            v_sh.at[pl.ds(sid * _RPS, _RPS)],
            out_hbm.at[pl.ds(cid * _NP + sid * _RPS, _RPS)])

    return sc_fn


def kernel(x, g_ij, idx_i, idx_j, n_atoms, rand_vec,
           W_g, b_g, W_j, b_j, W_i, b_i,
           int_res_W1, int_res_b1, int_res_W2, int_res_b2,
           W_f, b_f,
           atom_res_W1, atom_res_b1, atom_res_W2, atom_res_b2,
           out_res_W1, out_res_b1, out_res_W2, out_res_b2,
           W_out, b_out):
    n, d = x.shape
    e = g_ij.shape[0]
    r = g_ij.shape[1]
    del n_atoms

    row = lambda v: v.reshape(1, -1)

    u, vm = pl.pallas_call(
        _pre_body,
        grid=(n // _BN,),
        in_specs=[
            pl.BlockSpec((_BN, d), lambda i: (i, 0)),
            pl.BlockSpec((d, d), lambda i: (0, 0)),
            pl.BlockSpec((1, d), lambda i: (0, 0)),
            pl.BlockSpec((d, d), lambda i: (0, 0)),
            pl.BlockSpec((1, d), lambda i: (0, 0)),
        ],
        out_specs=[
            pl.BlockSpec((_BN, d), lambda i: (i, 0)),
            pl.BlockSpec((_BN, d), lambda i: (i, 0)),
        ],
        out_shape=[
            jax.ShapeDtypeStruct((n, d), jnp.float32),
            jax.ShapeDtypeStruct((n, d), jnp.float32),
        ],
    )(x, W_j, row(b_j), W_i, row(b_i))

    G = pl.pallas_call(
        _gmat_body,
        grid=(e // _BE,),
        in_specs=[
            pl.BlockSpec((_BE, r), lambda i: (i, 0)),
            pl.BlockSpec((r, d), lambda i: (0, 0)),
            pl.BlockSpec((1, d), lambda i: (0, 0)),
        ],
        out_specs=pl.BlockSpec((_BE, d), lambda i: (i, 0)),
        out_shape=jax.ShapeDtypeStruct((e, d), jnp.float32),
    )(g_ij, W_g, row(b_g))

    v_parts = _make_sc_kernel(n)(u, G, idx_i, idx_j)
    v0 = v_parts[:n]
    v1 = v_parts[_NP:_NP + n]

    n_int = int_res_W1.shape[0]
    n_atom = atom_res_W1.shape[0]
    n_out = out_res_W1.shape[0]

    mat_spec = pl.BlockSpec((d, d), lambda i: (0, 0))
    bias_spec = pl.BlockSpec((1, d), lambda i: (0, 0))
    blk_spec = pl.BlockSpec((_BN, d), lambda i: (i, 0))

    ops = [x, v0, v1, vm, row(rand_vec)]
    specs = [blk_spec, blk_spec, blk_spec, blk_spec, bias_spec]

    def add_res(W1, b1, W2, b2, k):
        ops.extend([W1[k], row(b1[k]), W2[k], row(b2[k])])
        specs.extend([mat_spec, bias_spec, mat_spec, bias_spec])

    for k in range(n_int):
        add_res(int_res_W1, int_res_b1, int_res_W2, int_res_b2, k)
    ops.extend([W_f, row(b_f)])
    specs.extend([mat_spec, bias_spec])
    for k in range(n_atom):
        add_res(atom_res_W1, atom_res_b1, atom_res_W2, atom_res_b2, k)
    for k in range(n_out):
        add_res(out_res_W1, out_res_b1, out_res_W2, out_res_b2, k)
    ops.extend([W_out, row(b_out)])
    specs.extend([mat_spec, bias_spec])

    out, x1 = pl.pallas_call(
        functools.partial(_post_body, n_int, n_atom, n_out),
        grid=(n // _BN,),
        in_specs=specs,
        out_specs=[blk_spec, blk_spec],
        out_shape=[
            jax.ShapeDtypeStruct((n, d), jnp.float32),
            jax.ShapeDtypeStruct((n, d), jnp.float32),
        ],
    )(*ops)
    return (out, x1)
